# Initial kernel scaffold; baseline (speedup 1.0000x reference)
import jax
import jax.numpy as jnp
from jax.experimental import pallas as pl


def kernel(x, edge_index, W_l, b_l, W_r, W_ln):
    raise NotImplementedError("write your pallas kernel here")


# speedup vs baseline: 4.7373x; 4.7373x over previous
import jax
import jax.numpy as jnp
from jax import lax
from jax.experimental import pallas as pl
from jax.experimental.pallas import tpu as pltpu
from jax.experimental.pallas import tpu_sc as plsc

N_NODES = 10000
N_EDGES = 320000
D = 128

NC = 2
NS = 16
LANES = 16
K = 80
E_PER_CORE = N_EDGES // NC
E_PER_TILE = E_PER_CORE // NS
N_CHUNKS = E_PER_TILE // K
N_PAD = 10240
ROWS_PER_TILE = N_PAD // NS
CW = 16
TC_BLK = 1000


def _sc_aggregate(x, src, dst):
    mesh = plsc.VectorSubcoreMesh(core_axis_name="c", subcore_axis_name="s")

    def body(x_h, src_h, dst_h, part_x_h,
             acc_x, idx_s, idx_d, rows, sem):
        c = lax.axis_index("c")
        s = lax.axis_index("s")
        r0 = s * ROWS_PER_TILE
        e0 = c * E_PER_CORE + s * E_PER_TILE
        out0 = c * N_PAD + r0

        zv = jnp.zeros((LANES,), jnp.float32)

        def zrow(i, carry):
            for l in range(D // LANES):
                rows[i, pl.ds(l * LANES, LANES)] = zv
            return carry

        lax.fori_loop(0, K, zrow, 0)

        for q in range(ROWS_PER_TILE // K):
            pltpu.sync_copy(rows, acc_x.at[pl.ds(r0 + q * K, K)])
        plsc.subcore_barrier()

        def chunk(j, carry):
            base = e0 + j * K
            pltpu.sync_copy(src_h.at[pl.ds(base, K)], idx_s)
            pltpu.sync_copy(dst_h.at[pl.ds(base, K)], idx_d)
            pltpu.async_copy(x_h.at[idx_s], rows, sem).wait()
            pltpu.sync_copy(rows, acc_x.at[idx_d], add=True)
            return carry

        lax.fori_loop(0, N_CHUNKS, chunk, 0)
        plsc.subcore_barrier()

        pltpu.sync_copy(acc_x.at[pl.ds(r0, ROWS_PER_TILE)],
                        part_x_h.at[pl.ds(out0, ROWS_PER_TILE)])


    call = pl.kernel(
        body,
        out_type=jax.ShapeDtypeStruct((NC * N_PAD, D), jnp.float32),
        mesh=mesh,
        scratch_types=[
            pltpu.VMEM_SHARED((N_PAD, D), jnp.float32),
            pltpu.VMEM((K,), jnp.int32),
            pltpu.VMEM((K,), jnp.int32),
            pltpu.VMEM((K, D), jnp.float32),
            pltpu.SemaphoreType.DMA,
        ],
    )
    return call(x, src, dst)


def _sc_count(dst):
    mesh = plsc.VectorSubcoreMesh(core_axis_name="c", subcore_axis_name="s")

    def body(dst_h, part_c_h, acc_c, idx_d, rows, ones_v, sem):
        c = lax.axis_index("c")
        s = lax.axis_index("s")
        r0 = s * ROWS_PER_TILE
        e0 = c * E_PER_CORE + s * E_PER_TILE
        out0 = c * N_PAD + r0

        zv = jnp.zeros((LANES,), jnp.float32)
        ov = jnp.ones((LANES,), jnp.float32)

        def zrow(i, carry):
            for l in range(D // LANES):
                rows[i, pl.ds(l * LANES, LANES)] = zv
                ones_v[i, pl.ds(l * LANES, LANES)] = ov
            return carry

        lax.fori_loop(0, K, zrow, 0)

        for q in range(ROWS_PER_TILE // K):
            pltpu.sync_copy(rows, acc_c.at[pl.ds(r0 + q * K, K)])
        plsc.subcore_barrier()

        def chunk(j, carry):
            base = e0 + j * K
            pltpu.sync_copy(dst_h.at[pl.ds(base, K)], idx_d)
            pltpu.sync_copy(ones_v, acc_c.at[idx_d], add=True)
            return carry

        lax.fori_loop(0, N_CHUNKS, chunk, 0)
        plsc.subcore_barrier()

        pltpu.sync_copy(acc_c.at[pl.ds(r0, ROWS_PER_TILE)],
                        part_c_h.at[pl.ds(out0, ROWS_PER_TILE)])

    call = pl.kernel(
        body,
        out_type=jax.ShapeDtypeStruct((NC * N_PAD, D), jnp.float32),
        mesh=mesh,
        scratch_types=[
            pltpu.VMEM_SHARED((N_PAD, D), jnp.float32),
            pltpu.VMEM((K,), jnp.int32),
            pltpu.VMEM((K, D), jnp.float32),
            pltpu.VMEM((K, D), jnp.float32),
            pltpu.SemaphoreType.DMA,
        ],
    )
    return call(dst)


def _tc_tail(px0, px1, pc0, pc1, x, WlT, bl, WrT, WlnT):
    def body(px0_ref, px1_ref, pc0_ref, pc1_ref, x_ref,
             wl_ref, bl_ref, wr_ref, wln_ref, o_ref):
        agg = px0_ref[...] + px1_ref[...]
        cnt = jnp.maximum(pc0_ref[...] + pc1_ref[...], 1.0)
        mean = agg / cnt
        xb = x_ref[...]
        h = jnp.dot(mean, wl_ref[...], preferred_element_type=jnp.float32)
        h = h + bl_ref[...] + jnp.dot(xb, wr_ref[...],
                                      preferred_element_type=jnp.float32)
        h = jnp.maximum(h, 0.0)
        o = jnp.dot(h, wln_ref[...], preferred_element_type=jnp.float32) + xb
        o_ref[...] = jnp.maximum(o, 0.0)

    grid = (N_NODES // TC_BLK,)
    return pl.pallas_call(
        body,
        grid=grid,
        in_specs=[
            pl.BlockSpec((TC_BLK, D), lambda i: (i, 0)),
            pl.BlockSpec((TC_BLK, D), lambda i: (i, 0)),
            pl.BlockSpec((TC_BLK, D), lambda i: (i, 0)),
            pl.BlockSpec((TC_BLK, D), lambda i: (i, 0)),
            pl.BlockSpec((TC_BLK, D), lambda i: (i, 0)),
            pl.BlockSpec((D, D), lambda i: (0, 0)),
            pl.BlockSpec((1, D), lambda i: (0, 0)),
            pl.BlockSpec((D, D), lambda i: (0, 0)),
            pl.BlockSpec((D, D), lambda i: (0, 0)),
        ],
        out_specs=pl.BlockSpec((TC_BLK, D), lambda i: (i, 0)),
        out_shape=jax.ShapeDtypeStruct((N_NODES, D), jnp.float32),
    )(px0, px1, pc0, pc1, x, WlT, bl, WrT, WlnT)


def kernel(x, edge_index, W_l, b_l, W_r, W_ln):
    src = edge_index[0].astype(jnp.int32)
    dst = edge_index[1].astype(jnp.int32)
    part_x = _sc_aggregate(x, src, dst)
    part_c = _sc_count(dst)
    px0, px1 = part_x[:N_PAD], part_x[N_PAD:]
    pc0, pc1 = part_c[:N_PAD], part_c[N_PAD:]
    return _tc_tail(px0, px1, pc0, pc1, x,
                    W_l.T, b_l.reshape(1, D), W_r.T, W_ln.T)


# speedup vs baseline: 6.4141x; 1.3540x over previous
import jax
import jax.numpy as jnp
from jax import lax
from jax.experimental import pallas as pl
from jax.experimental.pallas import tpu as pltpu
from jax.experimental.pallas import tpu_sc as plsc

N_NODES = 10000
N_EDGES = 320000
D = 128

NC = 2
NS = 16
LANES = 16
K = 80
E_PER_CORE = N_EDGES // NC
E_PER_TILE = E_PER_CORE // NS
N_CHUNKS = E_PER_TILE // K
N_PAD = 10240
ROWS_PER_TILE = N_PAD // NS
CW = 16
TC_BLK = 1000


def _sc_aggregate(x, src, dst):
    mesh = plsc.VectorSubcoreMesh(core_axis_name="c", subcore_axis_name="s")

    def body(x_h, src_h, dst_h, part_x_h,
             acc_x, idx_s0, idx_d0, rows0, idx_s1, idx_d1, rows1,
             sem0, sem1):
        c = lax.axis_index("c")
        s = lax.axis_index("s")
        r0 = s * ROWS_PER_TILE
        e0 = c * E_PER_CORE + s * E_PER_TILE
        out0 = c * N_PAD + r0

        zv = jnp.zeros((LANES,), jnp.float32)

        def zrow(i, carry):
            for l in range(D // LANES):
                rows0[i, pl.ds(l * LANES, LANES)] = zv
            return carry

        lax.fori_loop(0, K, zrow, 0)

        for q in range(ROWS_PER_TILE // K):
            pltpu.sync_copy(rows0, acc_x.at[pl.ds(r0 + q * K, K)])
        plsc.subcore_barrier()

        def load(j, idx_s, idx_d, rows, sem):
            base = e0 + j * K
            pltpu.sync_copy(src_h.at[pl.ds(base, K)], idx_s)
            pltpu.sync_copy(dst_h.at[pl.ds(base, K)], idx_d)
            pltpu.async_copy(x_h.at[idx_s], rows, sem)

        def drain_scatter(idx_s, idx_d, rows, sem):
            pltpu.make_async_copy(x_h.at[idx_s], rows, sem).wait()
            pltpu.sync_copy(rows, acc_x.at[idx_d], add=True)

        load(0, idx_s0, idx_d0, rows0, sem0)

        def body2(t, carry):
            load(2 * t + 1, idx_s1, idx_d1, rows1, sem1)
            drain_scatter(idx_s0, idx_d0, rows0, sem0)

            @pl.when(2 * t + 2 < N_CHUNKS)
            def _():
                load(2 * t + 2, idx_s0, idx_d0, rows0, sem0)

            drain_scatter(idx_s1, idx_d1, rows1, sem1)
            return carry

        lax.fori_loop(0, N_CHUNKS // 2, body2, 0)
        drain_scatter(idx_s0, idx_d0, rows0, sem0)
        plsc.subcore_barrier()

        pltpu.sync_copy(acc_x.at[pl.ds(r0, ROWS_PER_TILE)],
                        part_x_h.at[pl.ds(out0, ROWS_PER_TILE)])


    call = pl.kernel(
        body,
        out_type=jax.ShapeDtypeStruct((NC * N_PAD, D), jnp.float32),
        mesh=mesh,
        scratch_types=[
            pltpu.VMEM_SHARED((N_PAD, D), jnp.float32),
            pltpu.VMEM((K,), jnp.int32),
            pltpu.VMEM((K,), jnp.int32),
            pltpu.VMEM((K, D), jnp.float32),
            pltpu.VMEM((K,), jnp.int32),
            pltpu.VMEM((K,), jnp.int32),
            pltpu.VMEM((K, D), jnp.float32),
            pltpu.SemaphoreType.DMA,
            pltpu.SemaphoreType.DMA,
        ],
    )
    return call(x, src, dst)


def _sc_count(dst):
    mesh = plsc.VectorSubcoreMesh(core_axis_name="c", subcore_axis_name="s")

    def body(dst_h, part_c_h, acc_c, idx_d, rows, ones_v, sem):
        c = lax.axis_index("c")
        s = lax.axis_index("s")
        r0 = s * ROWS_PER_TILE
        e0 = c * E_PER_CORE + s * E_PER_TILE
        out0 = c * N_PAD + r0

        zv = jnp.zeros((LANES,), jnp.float32)
        ov = jnp.ones((LANES,), jnp.float32)

        def zrow(i, carry):
            for l in range(D // LANES):
                rows[i, pl.ds(l * LANES, LANES)] = zv
                ones_v[i, pl.ds(l * LANES, LANES)] = ov
            return carry

        lax.fori_loop(0, K, zrow, 0)

        for q in range(ROWS_PER_TILE // K):
            pltpu.sync_copy(rows, acc_c.at[pl.ds(r0 + q * K, K)])
        plsc.subcore_barrier()

        def chunk(j, carry):
            base = e0 + j * K
            pltpu.sync_copy(dst_h.at[pl.ds(base, K)], idx_d)
            pltpu.sync_copy(ones_v, acc_c.at[idx_d], add=True)
            return carry

        lax.fori_loop(0, N_CHUNKS, chunk, 0)
        plsc.subcore_barrier()

        pltpu.sync_copy(acc_c.at[pl.ds(r0, ROWS_PER_TILE)],
                        part_c_h.at[pl.ds(out0, ROWS_PER_TILE)])

    call = pl.kernel(
        body,
        out_type=jax.ShapeDtypeStruct((NC * N_PAD, D), jnp.float32),
        mesh=mesh,
        scratch_types=[
            pltpu.VMEM_SHARED((N_PAD, D), jnp.float32),
            pltpu.VMEM((K,), jnp.int32),
            pltpu.VMEM((K, D), jnp.float32),
            pltpu.VMEM((K, D), jnp.float32),
            pltpu.SemaphoreType.DMA,
        ],
    )
    return call(dst)


def _tc_tail(px0, px1, pc0, pc1, x, WlT, bl, WrT, WlnT):
    def body(px0_ref, px1_ref, pc0_ref, pc1_ref, x_ref,
             wl_ref, bl_ref, wr_ref, wln_ref, o_ref):
        agg = px0_ref[...] + px1_ref[...]
        cnt = jnp.maximum(pc0_ref[...] + pc1_ref[...], 1.0)
        mean = agg / cnt
        xb = x_ref[...]
        h = jnp.dot(mean, wl_ref[...], preferred_element_type=jnp.float32)
        h = h + bl_ref[...] + jnp.dot(xb, wr_ref[...],
                                      preferred_element_type=jnp.float32)
        h = jnp.maximum(h, 0.0)
        o = jnp.dot(h, wln_ref[...], preferred_element_type=jnp.float32) + xb
        o_ref[...] = jnp.maximum(o, 0.0)

    grid = (N_NODES // TC_BLK,)
    return pl.pallas_call(
        body,
        grid=grid,
        in_specs=[
            pl.BlockSpec((TC_BLK, D), lambda i: (i, 0)),
            pl.BlockSpec((TC_BLK, D), lambda i: (i, 0)),
            pl.BlockSpec((TC_BLK, D), lambda i: (i, 0)),
            pl.BlockSpec((TC_BLK, D), lambda i: (i, 0)),
            pl.BlockSpec((TC_BLK, D), lambda i: (i, 0)),
            pl.BlockSpec((D, D), lambda i: (0, 0)),
            pl.BlockSpec((1, D), lambda i: (0, 0)),
            pl.BlockSpec((D, D), lambda i: (0, 0)),
            pl.BlockSpec((D, D), lambda i: (0, 0)),
        ],
        out_specs=pl.BlockSpec((TC_BLK, D), lambda i: (i, 0)),
        out_shape=jax.ShapeDtypeStruct((N_NODES, D), jnp.float32),
    )(px0, px1, pc0, pc1, x, WlT, bl, WrT, WlnT)


def kernel(x, edge_index, W_l, b_l, W_r, W_ln):
    src = edge_index[0].astype(jnp.int32)
    dst = edge_index[1].astype(jnp.int32)
    part_x = _sc_aggregate(x, src, dst)
    part_c = _sc_count(dst)
    px0, px1 = part_x[:N_PAD], part_x[N_PAD:]
    pc0, pc1 = part_c[:N_PAD], part_c[N_PAD:]
    return _tc_tail(px0, px1, pc0, pc1, x,
                    W_l.T, b_l.reshape(1, D), W_r.T, W_ln.T)
